# baseline (speedup 1.0000x reference)
import functools

import jax
import jax.numpy as jnp
from jax import lax
from jax.experimental import pallas as pl
from jax.experimental.pallas import tpu as pltpu
from jax.experimental.pallas import tpu_sc as plsc

_B, _C = 2, 96
_OH = _OW = 27
_KH = _KW = 16
_H = _W = 224
_N_IMG = _B * _C
_ROW_ELEMS = _OW * _KH * _KW
_IMG_OUT = _H * _W
_N_WORKERS = 32
_IMGS_PER_WORKER = _N_IMG // _N_WORKERS
_CHROWS = 3
_NCH = _OH // _CHROWS
_CHUNK = _CHROWS * _ROW_ELEMS


def _fold_sc(xr):
    mesh = plsc.VectorSubcoreMesh(core_axis_name="c", subcore_axis_name="s")

    @functools.partial(
        pl.kernel,
        out_type=jax.ShapeDtypeStruct((_N_IMG * _IMG_OUT,), jnp.float32),
        mesh=mesh,
        scratch_types=[
            pltpu.VMEM((_CHUNK,), jnp.float32),
            pltpu.VMEM((_CHUNK,), jnp.float32),
            pltpu.VMEM((_CHUNK,), jnp.float32),
            pltpu.VMEM((_IMG_OUT,), jnp.float32),
            pltpu.SemaphoreType.DMA,
            pltpu.SemaphoreType.DMA,
            pltpu.SemaphoreType.DMA,
            pltpu.SemaphoreType.DMA,
        ],
    )
    def k(x_hbm, out_hbm, rb0, rb1, rb2, obuf, s0, s1, s2, so):
        wid = lax.axis_index("s") * 2 + lax.axis_index("c")
        zeros16 = jnp.zeros((16,), jnp.float32)
        sems = [s0, s1, s2]
        rbufs = [rb0, rb1, rb2]

        def wait_in(slot):
            pltpu.make_async_copy(
                x_hbm.at[pl.ds(0, _CHUNK)], rbufs[slot], sems[slot]
            ).wait()

        def wait_out():
            pltpu.make_async_copy(obuf, out_hbm.at[pl.ds(0, _IMG_OUT)], so).wait()

        def compute_chunk(ch, slot):
            def kh_body(kh, carry):
                for r in range(_CHROWS):
                    dst_base = (8 * (ch * _CHROWS + r) + kh) * _W
                    src_base = r * _ROW_ELEMS + kh * _KW
                    for j in range(_OW):
                        v = rbufs[slot][pl.ds(src_base + j * (_KH * _KW), 16)]
                        plsc.addupdate(obuf.at[pl.ds(dst_base + 8 * j, 16)], v)
                return carry

            lax.fori_loop(0, _KH, kh_body, 0)

        def zero_body(t, carry):
            for u in range(_W // 16):
                obuf[pl.ds(t * _W + u * 16, 16)] = zeros16
            return carry

        def image_body(m, carry):
            img = wid * _IMGS_PER_WORKER + m
            ibase = img * (_NCH * _CHUNK)

            @pl.when(m > 0)
            def _():
                wait_out()

            lax.fori_loop(0, _H, zero_body, 0)

            def g_body(ch, inner):
                pltpu.sync_copy(
                    x_hbm.at[pl.ds(ibase + ch * _CHUNK, _CHUNK)], rb0
                )
                compute_chunk(ch, 0)
                return inner

            lax.fori_loop(0, _NCH, g_body, 0)
            pltpu.async_copy(obuf, out_hbm.at[pl.ds(img * _IMG_OUT, _IMG_OUT)], so)
            return carry

        lax.fori_loop(0, _IMGS_PER_WORKER, image_body, 0)
        wait_out()

    return k(xr)


def kernel(x):
    xr = x.reshape(_N_IMG * _NCH * _CHUNK)
    out = _fold_sc(xr)
    return out.reshape(_B, _C, _H, _W)


# speedup vs baseline: 1.1908x; 1.1908x over previous
import functools

import jax
import jax.numpy as jnp
from jax import lax
from jax.experimental import pallas as pl
from jax.experimental.pallas import tpu as pltpu
from jax.experimental.pallas import tpu_sc as plsc

_B, _C = 2, 96
_OH = _OW = 27
_KH = _KW = 16
_H = _W = 224
_N_IMG = _B * _C
_ROW_ELEMS = _OW * _KH * _KW
_IMG_OUT = _H * _W
_N_WORKERS = 32
_IMGS_PER_WORKER = _N_IMG // _N_WORKERS
_CHROWS = 3
_NCH = _OH // _CHROWS
_CHUNK = _CHROWS * _ROW_ELEMS


def _fold_sc(xr):
    mesh = plsc.VectorSubcoreMesh(core_axis_name="c", subcore_axis_name="s")

    @functools.partial(
        pl.kernel,
        out_type=jax.ShapeDtypeStruct((_N_IMG * _IMG_OUT,), jnp.float32),
        mesh=mesh,
        scratch_types=[
            pltpu.VMEM((_CHUNK,), jnp.float32),
            pltpu.VMEM((_CHUNK,), jnp.float32),
            pltpu.VMEM((_CHUNK,), jnp.float32),
            pltpu.VMEM((_IMG_OUT,), jnp.float32),
            pltpu.SemaphoreType.DMA,
            pltpu.SemaphoreType.DMA,
            pltpu.SemaphoreType.DMA,
            pltpu.SemaphoreType.DMA,
        ],
    )
    def k(x_hbm, out_hbm, rb0, rb1, rb2, obuf, s0, s1, s2, so):
        wid = lax.axis_index("s") * 2 + lax.axis_index("c")
        zeros16 = jnp.zeros((16,), jnp.float32)
        sems = [s0, s1, s2]
        rbufs = [rb0, rb1, rb2]

        def wait_in(slot):
            pltpu.make_async_copy(
                x_hbm.at[pl.ds(0, _CHUNK)], rbufs[slot], sems[slot]
            ).wait()

        def wait_out():
            pltpu.make_async_copy(obuf, out_hbm.at[pl.ds(0, _IMG_OUT)], so).wait()

        def compute_chunk(ch, slot):
            def kh_body(kh, carry):
                for r in range(_CHROWS):
                    dst_base = (8 * (ch * _CHROWS + r) + kh) * _W
                    src_base = r * _ROW_ELEMS + kh * _KW
                    vs = [
                        rbufs[slot][pl.ds(src_base + j * (_KH * _KW), 16)]
                        for j in range(_OW)
                    ]
                    for j in range(_OW):
                        plsc.addupdate(obuf.at[pl.ds(dst_base + 8 * j, 16)], vs[j])
                return carry

            lax.fori_loop(0, _KH, kh_body, 0)

        def zero_body(t, carry):
            for u in range(_W // 16):
                obuf[pl.ds(t * _W + u * 16, 16)] = zeros16
            return carry

        def image_body(m, carry):
            img = wid * _IMGS_PER_WORKER + m
            ibase = img * (_NCH * _CHUNK)
            pltpu.async_copy(x_hbm.at[pl.ds(ibase, _CHUNK)], rb0, s0)
            pltpu.async_copy(x_hbm.at[pl.ds(ibase + _CHUNK, _CHUNK)], rb1, s1)

            @pl.when(m > 0)
            def _():
                wait_out()

            lax.fori_loop(0, _H, zero_body, 0)

            def g_body(g, inner):
                for s in range(3):
                    ch = 3 * g + s
                    wait_in(s)
                    compute_chunk(ch, s)
                    nslot = (s + 2) % 3

                    @pl.when(ch + 2 <= _NCH - 1)
                    def _():
                        pltpu.async_copy(
                            x_hbm.at[pl.ds(ibase + (ch + 2) * _CHUNK, _CHUNK)],
                            rbufs[nslot],
                            sems[nslot],
                        )

                return inner

            lax.fori_loop(0, _NCH // 3, g_body, 0)
            pltpu.async_copy(obuf, out_hbm.at[pl.ds(img * _IMG_OUT, _IMG_OUT)], so)
            return carry

        lax.fori_loop(0, _IMGS_PER_WORKER, image_body, 0)
        wait_out()

    return k(xr)


def kernel(x):
    xr = x.reshape(_N_IMG * _NCH * _CHUNK)
    out = _fold_sc(xr)
    return out.reshape(_B, _C, _H, _W)
